# baseline (speedup 1.0000x reference)
import jax
import jax.numpy as jnp
from jax.experimental import pallas as pl
from jax.experimental.pallas import tpu as pltpu

B, D, H, L, K = 4096, 768, 1024, 256, 1024
TB = 512
GRID = B // TB


_SQRT_HALF = 0.7071067811865476


def _gelu(v):
    return 0.5 * v * (1.0 + jax.lax.erf(v * _SQRT_HALF))


def _fused_kernel(x_ref, embed_ref,
                  ew1_ref, eb1_ref, ew2_ref, eb2_ref, ew3_ref, eb3_ref,
                  dw1_ref, db1_ref, dw2_ref, db2_ref, dw3_ref, db3_ref,
                  xp_ref, zd_ref, loss_ref):
    i = pl.program_id(0)
    x = x_ref[...]
    embed = embed_ref[...]
    e2 = jnp.sum(embed * embed, axis=1)[None, :]

    h = _gelu(jnp.dot(x, ew1_ref[...], preferred_element_type=jnp.float32)
              + eb1_ref[...])
    h = _gelu(jnp.dot(h, ew2_ref[...], preferred_element_type=jnp.float32)
              + eb2_ref[...])
    z_e = (jnp.dot(h, ew3_ref[...], preferred_element_type=jnp.float32)
           + eb3_ref[...])

    ze2 = jnp.sum(z_e * z_e, axis=1, keepdims=True)
    cross = jax.lax.dot_general(
        z_e, embed, (((1,), (1,)), ((), ())),
        preferred_element_type=jnp.float32)
    d2 = ze2 + e2 - 2.0 * cross

    m = jnp.min(d2, axis=1, keepdims=True)
    iota = jax.lax.broadcasted_iota(jnp.int32, (TB, K), 1)
    idx = jnp.min(jnp.where(d2 == m, iota, K), axis=1, keepdims=True)
    onehot = (iota == idx).astype(jnp.float32)
    zd_ref[...] = onehot.astype(jnp.int32)

    z_q = jnp.dot(onehot, embed, preferred_element_type=jnp.float32)

    g = _gelu(jnp.dot(z_q.astype(jnp.bfloat16), dw1_ref[...],
                      preferred_element_type=jnp.float32)
              + db1_ref[...])
    g = _gelu(jnp.dot(g.astype(jnp.bfloat16), dw2_ref[...],
                      preferred_element_type=jnp.float32)
              + db2_ref[...])
    x_pred = jax.nn.sigmoid(
        jnp.dot(g.astype(jnp.bfloat16), dw3_ref[...],
                preferred_element_type=jnp.float32)
        + db3_ref[...])
    xp_ref[...] = x_pred

    dxe = x - x_pred
    dzq = z_e - z_q
    partial = ((jnp.sum(dxe * dxe) + 1.25 * jnp.sum(dzq * dzq))
               * (1.0 / B)).reshape(1, 1)

    @pl.when(i == 0)
    def _init():
        loss_ref[...] = partial

    @pl.when(i != 0)
    def _acc():
        loss_ref[...] += partial


def _full(shape):
    return pl.BlockSpec(shape, lambda i: tuple(0 for _ in shape))


@jax.jit
def kernel(x, embed, enc_w1, enc_b1, enc_w2, enc_b2, enc_w3, enc_b3,
           dec_w1, dec_b1, dec_w2, dec_b2, dec_w3, dec_b3):
    biases = [b.reshape(1, -1) for b in
              (enc_b1, enc_b2, enc_b3, dec_b1, dec_b2, dec_b3)]
    eb1, eb2, eb3, db1, db2, db3 = biases
    dec_w1 = dec_w1.astype(jnp.bfloat16)
    dec_w2 = dec_w2.astype(jnp.bfloat16)
    dec_w3 = dec_w3.astype(jnp.bfloat16)

    grid_spec = pl.GridSpec(
        grid=(GRID,),
        in_specs=[
            pl.BlockSpec((TB, D), lambda i: (i, 0)),
            _full((K, L)),
            _full((D, H)), _full((1, H)),
            _full((H, H)), _full((1, H)),
            _full((H, L)), _full((1, L)),
            _full((L, H)), _full((1, H)),
            _full((H, H)), _full((1, H)),
            _full((H, D)), _full((1, D)),
        ],
        out_specs=[
            pl.BlockSpec((TB, D), lambda i: (i, 0)),
            pl.BlockSpec((TB, K), lambda i: (i, 0)),
            pl.BlockSpec((1, 1), lambda i: (0, 0)),
        ],
    )
    x_pred, z_disc, loss = pl.pallas_call(
        _fused_kernel,
        grid_spec=grid_spec,
        out_shape=[
            jax.ShapeDtypeStruct((B, D), jnp.float32),
            jax.ShapeDtypeStruct((B, K), jnp.int32),
            jax.ShapeDtypeStruct((1, 1), jnp.float32),
        ],
        compiler_params=pltpu.CompilerParams(
            dimension_semantics=("arbitrary",),
        ),
    )(x, embed, enc_w1, eb1, enc_w2, eb2, enc_w3, eb3,
      dec_w1, db1, dec_w2, db2, dec_w3, db3)
    return (x_pred, z_disc, loss[0, 0])


# speedup vs baseline: 1.0897x; 1.0897x over previous
import jax
import jax.numpy as jnp
from jax.experimental import pallas as pl
from jax.experimental.pallas import tpu as pltpu

B, D, H, L, K = 4096, 768, 1024, 256, 1024
TB = 512
GRID = B // TB


_SQRT_HALF = 0.7071067811865476


def _gelu(v):
    return 0.5 * v * (1.0 + jax.lax.erf(v * _SQRT_HALF))


def _fused_kernel(x_ref, embed_ref,
                  ew1_ref, eb1_ref, ew2_ref, eb2_ref, ew3_ref, eb3_ref,
                  dw1_ref, db1_ref, dw2_ref, db2_ref, dw3_ref, db3_ref,
                  xp_ref, zd_ref, loss_ref):
    i = pl.program_id(0)
    x = x_ref[...]
    embed = embed_ref[...]
    e2 = jnp.sum(embed * embed, axis=1)[None, :]

    h = _gelu(jnp.dot(x, ew1_ref[...], preferred_element_type=jnp.float32)
              + eb1_ref[...])
    h = _gelu(jnp.dot(h, ew2_ref[...], preferred_element_type=jnp.float32)
              + eb2_ref[...])
    z_e = (jnp.dot(h, ew3_ref[...], preferred_element_type=jnp.float32)
           + eb3_ref[...])

    ze2 = jnp.sum(z_e * z_e, axis=1, keepdims=True)
    cross = jax.lax.dot_general(
        z_e, embed, (((1,), (1,)), ((), ())),
        preferred_element_type=jnp.float32)
    d2 = ze2 + e2 - 2.0 * cross

    m = jnp.min(d2, axis=1, keepdims=True)
    iota = jax.lax.broadcasted_iota(jnp.int32, (TB, K), 1)
    idx = jnp.min(jnp.where(d2 == m, iota, K), axis=1, keepdims=True)
    onehot = (iota == idx).astype(jnp.float32)
    zd_ref[...] = onehot.astype(jnp.int32)

    z_q = jnp.dot(onehot, embed, preferred_element_type=jnp.float32)

    g = _gelu(jnp.dot(z_q, dw1_ref[...], preferred_element_type=jnp.float32)
              + db1_ref[...])
    g = _gelu(jnp.dot(g, dw2_ref[...], preferred_element_type=jnp.float32)
              + db2_ref[...])
    x_pred = jax.nn.sigmoid(
        jnp.dot(g, dw3_ref[...], preferred_element_type=jnp.float32)
        + db3_ref[...])
    xp_ref[...] = x_pred

    dxe = x - x_pred
    dzq = z_e - z_q
    partial = ((jnp.sum(dxe * dxe) + 1.25 * jnp.sum(dzq * dzq))
               * (1.0 / B)).reshape(1, 1)

    @pl.when(i == 0)
    def _init():
        loss_ref[...] = partial

    @pl.when(i != 0)
    def _acc():
        loss_ref[...] += partial


def _full(shape):
    return pl.BlockSpec(shape, lambda i: tuple(0 for _ in shape))


@jax.jit
def kernel(x, embed, enc_w1, enc_b1, enc_w2, enc_b2, enc_w3, enc_b3,
           dec_w1, dec_b1, dec_w2, dec_b2, dec_w3, dec_b3):
    biases = [b.reshape(1, -1) for b in
              (enc_b1, enc_b2, enc_b3, dec_b1, dec_b2, dec_b3)]
    eb1, eb2, eb3, db1, db2, db3 = biases

    grid_spec = pl.GridSpec(
        grid=(GRID,),
        in_specs=[
            pl.BlockSpec((TB, D), lambda i: (i, 0)),
            _full((K, L)),
            _full((D, H)), _full((1, H)),
            _full((H, H)), _full((1, H)),
            _full((H, L)), _full((1, L)),
            _full((L, H)), _full((1, H)),
            _full((H, H)), _full((1, H)),
            _full((H, D)), _full((1, D)),
        ],
        out_specs=[
            pl.BlockSpec((TB, D), lambda i: (i, 0)),
            pl.BlockSpec((TB, K), lambda i: (i, 0)),
            pl.BlockSpec((1, 1), lambda i: (0, 0)),
        ],
    )
    x_pred, z_disc, loss = pl.pallas_call(
        _fused_kernel,
        grid_spec=grid_spec,
        out_shape=[
            jax.ShapeDtypeStruct((B, D), jnp.float32),
            jax.ShapeDtypeStruct((B, K), jnp.int32),
            jax.ShapeDtypeStruct((1, 1), jnp.float32),
        ],
        compiler_params=pltpu.CompilerParams(
            dimension_semantics=("arbitrary",),
        ),
    )(x, embed, enc_w1, eb1, enc_w2, eb2, enc_w3, eb3,
      dec_w1, db1, dec_w2, db2, dec_w3, db3)
    return (x_pred, z_disc, loss[0, 0])


# speedup vs baseline: 1.1342x; 1.0408x over previous
import jax
import jax.numpy as jnp
from jax.experimental import pallas as pl
from jax.experimental.pallas import tpu as pltpu

B, D, H, L, K = 4096, 768, 1024, 256, 1024
TB = 1024
GRID = B // TB


_SQRT_HALF = 0.7071067811865476


def _gelu(v):
    return 0.5 * v * (1.0 + jax.lax.erf(v * _SQRT_HALF))


def _fused_kernel(x_ref, embed_ref,
                  ew1_ref, eb1_ref, ew2_ref, eb2_ref, ew3_ref, eb3_ref,
                  dw1_ref, db1_ref, dw2_ref, db2_ref, dw3_ref, db3_ref,
                  xp_ref, zd_ref, loss_ref):
    i = pl.program_id(0)
    x = x_ref[...]
    embed = embed_ref[...]
    e2 = jnp.sum(embed * embed, axis=1)[None, :]

    h = _gelu(jnp.dot(x, ew1_ref[...], preferred_element_type=jnp.float32)
              + eb1_ref[...])
    h = _gelu(jnp.dot(h, ew2_ref[...], preferred_element_type=jnp.float32)
              + eb2_ref[...])
    z_e = (jnp.dot(h, ew3_ref[...], preferred_element_type=jnp.float32)
           + eb3_ref[...])

    ze2 = jnp.sum(z_e * z_e, axis=1, keepdims=True)
    cross = jax.lax.dot_general(
        z_e, embed, (((1,), (1,)), ((), ())),
        preferred_element_type=jnp.float32)
    d2 = ze2 + e2 - 2.0 * cross

    m = jnp.min(d2, axis=1, keepdims=True)
    iota = jax.lax.broadcasted_iota(jnp.int32, (TB, K), 1)
    idx = jnp.min(jnp.where(d2 == m, iota, K), axis=1, keepdims=True)
    onehot = (iota == idx).astype(jnp.float32)
    zd_ref[...] = onehot.astype(jnp.int32)

    z_q = jnp.dot(onehot, embed, preferred_element_type=jnp.float32)

    g = _gelu(jnp.dot(z_q, dw1_ref[...], preferred_element_type=jnp.float32)
              + db1_ref[...])
    g = _gelu(jnp.dot(g, dw2_ref[...], preferred_element_type=jnp.float32)
              + db2_ref[...])
    x_pred = jax.nn.sigmoid(
        jnp.dot(g, dw3_ref[...], preferred_element_type=jnp.float32)
        + db3_ref[...])
    xp_ref[...] = x_pred

    dxe = x - x_pred
    dzq = z_e - z_q
    partial = ((jnp.sum(dxe * dxe) + 1.25 * jnp.sum(dzq * dzq))
               * (1.0 / B)).reshape(1, 1)

    @pl.when(i == 0)
    def _init():
        loss_ref[...] = partial

    @pl.when(i != 0)
    def _acc():
        loss_ref[...] += partial


def _full(shape):
    return pl.BlockSpec(shape, lambda i: tuple(0 for _ in shape))


@jax.jit
def kernel(x, embed, enc_w1, enc_b1, enc_w2, enc_b2, enc_w3, enc_b3,
           dec_w1, dec_b1, dec_w2, dec_b2, dec_w3, dec_b3):
    biases = [b.reshape(1, -1) for b in
              (enc_b1, enc_b2, enc_b3, dec_b1, dec_b2, dec_b3)]
    eb1, eb2, eb3, db1, db2, db3 = biases

    grid_spec = pl.GridSpec(
        grid=(GRID,),
        in_specs=[
            pl.BlockSpec((TB, D), lambda i: (i, 0)),
            _full((K, L)),
            _full((D, H)), _full((1, H)),
            _full((H, H)), _full((1, H)),
            _full((H, L)), _full((1, L)),
            _full((L, H)), _full((1, H)),
            _full((H, H)), _full((1, H)),
            _full((H, D)), _full((1, D)),
        ],
        out_specs=[
            pl.BlockSpec((TB, D), lambda i: (i, 0)),
            pl.BlockSpec((TB, K), lambda i: (i, 0)),
            pl.BlockSpec((1, 1), lambda i: (0, 0)),
        ],
    )
    x_pred, z_disc, loss = pl.pallas_call(
        _fused_kernel,
        grid_spec=grid_spec,
        out_shape=[
            jax.ShapeDtypeStruct((B, D), jnp.float32),
            jax.ShapeDtypeStruct((B, K), jnp.int32),
            jax.ShapeDtypeStruct((1, 1), jnp.float32),
        ],
        compiler_params=pltpu.CompilerParams(
            dimension_semantics=("arbitrary",),
        ),
    )(x, embed, enc_w1, eb1, enc_w2, eb2, enc_w3, eb3,
      dec_w1, db1, dec_w2, db2, dec_w3, db3)
    return (x_pred, z_disc, loss[0, 0])
